# baseline (speedup 1.0000x reference)
import jax
import jax.numpy as jnp
from jax import lax
from jax.experimental import pallas as pl
from jax.experimental.pallas import tpu as pltpu
from jax.experimental.pallas import tpu_sc as plsc

DIM = 32
NC = 2
NS = 16
NW = NC * NS
L = 16
GATHER = 128


def _trans_body(tab_t_hbm, tail_hbm, scr_hbm, in_v, out_v,
                semi0, semi1, semo0, semo1):
    n = tab_t_hbm.shape[1]
    nblk_full = n // 128
    rem = n - nblk_full * 128
    wid = lax.axis_index("s") * NC + lax.axis_index("c")
    npairs = nblk_full // NW // 2
    extra_w = nblk_full - npairs * 2 * NW

    sems_i = (semi0, semi1)
    sems_o = (semo0, semo1)

    def blk(p, buf):
        return wid + (2 * p + buf) * NW

    def start_in(t, buf):
        pltpu.async_copy(tab_t_hbm.at[:, pl.ds(t * 128, 128)], in_v.at[buf],
                         sems_i[buf])

    def drain_in(t, buf):
        pltpu.make_async_copy(tab_t_hbm.at[:, pl.ds(t * 128, 128)],
                              in_v.at[buf], sems_i[buf]).wait()

    def transpose(buf, q0, nq):
        src = in_v.at[buf]
        half = jnp.arange(L, dtype=jnp.int32)

        for q in range(q0, q0 + nq):
            for jj in range(4):
                col = jnp.full((L,), q * 4 + jj, jnp.int32)
                out_v[buf, q, pl.ds(jj * DIM, L)] = plsc.load_gather(
                    src, [half, col])
                out_v[buf, q, pl.ds(jj * DIM + L, L)] = plsc.load_gather(
                    src, [half + L, col])

    def start_out(t, buf, nrows):
        pltpu.async_copy(out_v.at[buf, pl.ds(0, nrows)],
                         scr_hbm.at[pl.ds(t * 32, nrows)], sems_o[buf])

    def wait_out(t, buf, nrows):
        pltpu.make_async_copy(out_v.at[buf, pl.ds(0, nrows)],
                              scr_hbm.at[pl.ds(t * 32, nrows)],
                              sems_o[buf]).wait()

    start_in(blk(0, 0), 0)

    def pair(p, carry):
        t0 = blk(p, 0)
        t1 = blk(p, 1)
        start_in(t1, 1)
        drain_in(t0, 0)

        @pl.when(p >= 1)
        def _():
            wait_out(blk(p - 1, 0), 0, 32)

        transpose(0, 0, 32)
        start_out(t0, 0, 32)

        @pl.when(p + 1 < npairs)
        def _():
            start_in(blk(p + 1, 0), 0)

        drain_in(t1, 1)

        @pl.when(p >= 1)
        def _():
            wait_out(blk(p - 1, 1), 1, 32)

        transpose(1, 0, 32)
        start_out(t1, 1, 32)
        return carry

    lax.fori_loop(0, npairs, pair, 0)
    wait_out(blk(npairs - 1, 0), 0, 32)
    wait_out(blk(npairs - 1, 1), 1, 32)

    @pl.when(wid < extra_w)
    def _():
        t = npairs * 2 * NW + wid
        pltpu.sync_copy(tab_t_hbm.at[:, pl.ds(t * 128, 128)], in_v.at[0])
        transpose(0, 0, 32)
        pltpu.sync_copy(out_v.at[0], scr_hbm.at[pl.ds(t * 32, 32)])

    if rem:
        @pl.when(wid == extra_w)
        def _():
            pltpu.sync_copy(tail_hbm, in_v.at[0])
            transpose(0, 0, rem // 4)
            pltpu.sync_copy(out_v.at[0, pl.ds(0, rem // 4)],
                            scr_hbm.at[pl.ds(nblk_full * 32, rem // 4)])


def _make_trans(n):
    assert (n * DIM) % 128 == 0
    return pl.kernel(
        _trans_body,
        out_type=jax.ShapeDtypeStruct((n * DIM // 128, 128), jnp.float32),
        mesh=plsc.VectorSubcoreMesh(core_axis_name="c", subcore_axis_name="s"),
        scratch_types=[
            pltpu.VMEM((2, DIM, 128), jnp.float32),
            pltpu.VMEM((2, DIM, 128), jnp.float32),
            pltpu.SemaphoreType.DMA,
            pltpu.SemaphoreType.DMA,
            pltpu.SemaphoreType.DMA,
            pltpu.SemaphoreType.DMA,
        ],
        compiler_params=pltpu.CompilerParams(
            use_tc_tiling_on_sc=True, needs_layout_passes=False),
    )


def _gather_body(tab_hbm, xt_hbm, out_hbm, idxf_v, srow_v, scol_v, rows_v,
                 tile_v, semi, semg0, semg1, semo0, semo1):
    seq = xt_hbm.shape[0]
    bpw = xt_hbm.shape[1] // NW
    upw = seq * (bpw // GATHER)
    wid = lax.axis_index("s") * NC + lax.axis_index("c")
    b0 = wid * bpw

    for s in range(seq):
        pltpu.async_copy(xt_hbm.at[s, pl.ds(b0, bpw)],
                         idxf_v.at[pl.ds(s * bpw, bpw)], semi)
    for s in range(seq):
        pltpu.make_async_copy(xt_hbm.at[s, pl.ds(b0, bpw)],
                              idxf_v.at[pl.ds(s * bpw, bpw)], semi).wait()

    sems_g = (semg0, semg1)
    sems_o = (semo0, semo1)
    nbsub = bpw // GATHER

    def prep(u, buf):
        base = u * GATHER
        for v in range(GATHER // L):
            iv = idxf_v[pl.ds(base + v * L, L)]
            srow_v[buf, pl.ds(v * L, L)] = lax.shift_right_logical(iv, 2)
            scol_v[buf, pl.ds(v * L, L)] = (iv & 3) * DIM

    def fire(buf):
        pltpu.async_copy(tab_hbm.at[srow_v.at[buf]], rows_v.at[buf],
                         sems_g[buf])

    def drain_gather(buf):
        pltpu.make_async_copy(tab_hbm.at[pl.ds(0, GATHER)], rows_v.at[buf],
                              sems_g[buf]).wait()

    def extract(buf):
        rows = rows_v.at[buf]
        cols = [scol_v[buf, pl.ds(v * L, L)] for v in range(GATHER // L)]
        segs = [jnp.arange(v * L, v * L + L, dtype=jnp.int32)
                for v in range(GATHER // L)]
        for c in range(DIM):
            for v in range(GATHER // L):
                tile_v[buf, c, pl.ds(v * L, L)] = plsc.load_gather(
                    rows, [segs[v], cols[v] + c])

    def out_slice(u):
        s = lax.div(u, nbsub)
        bg = b0 + lax.rem(u, nbsub) * GATHER
        return out_hbm.at[s, :, pl.ds(bg, GATHER)]

    def wait_writeout(buf, u):
        pltpu.make_async_copy(tile_v.at[buf], out_slice(u), sems_o[buf]
                              ).wait()

    prep(0, 0)
    fire(0)

    def pair(p, carry):
        u0 = 2 * p
        u1 = 2 * p + 1

        prep(u1, 1)
        drain_gather(0)
        fire(1)

        @pl.when(p >= 1)
        def _():
            wait_writeout(0, u0 - 2)

        extract(0)
        pltpu.async_copy(tile_v.at[0], out_slice(u0), sems_o[0])

        @pl.when(p + 1 < upw // 2)
        def _():
            prep(u0 + 2, 0)
            fire(0)

        drain_gather(1)

        @pl.when(p >= 1)
        def _():
            wait_writeout(1, u1 - 2)

        extract(1)
        pltpu.async_copy(tile_v.at[1], out_slice(u1), sems_o[1])
        return carry

    lax.fori_loop(0, upw // 2, pair, 0)
    wait_writeout(0, upw - 2)
    wait_writeout(1, upw - 1)


def _make_gather(seq, b):
    return pl.kernel(
        _gather_body,
        out_type=jax.ShapeDtypeStruct((seq, DIM, b), jnp.float32),
        mesh=plsc.VectorSubcoreMesh(core_axis_name="c", subcore_axis_name="s"),
        scratch_types=[
            pltpu.VMEM((seq * (b // NW),), jnp.int32),
            pltpu.VMEM((2, GATHER), jnp.int32),
            pltpu.VMEM((2, GATHER), jnp.int32),
            pltpu.VMEM((2, GATHER, 128), jnp.float32),
            pltpu.VMEM((2, DIM, GATHER), jnp.float32),
            pltpu.SemaphoreType.DMA,
            pltpu.SemaphoreType.DMA,
            pltpu.SemaphoreType.DMA,
            pltpu.SemaphoreType.DMA,
            pltpu.SemaphoreType.DMA,
        ],
        compiler_params=pltpu.CompilerParams(
            use_tc_tiling_on_sc=True, needs_layout_passes=False),
    )


def kernel(x, table):
    b, seq = x.shape
    n, d = table.shape
    assert d == DIM and (n * d) % 128 == 0
    xt = x.T.astype(jnp.int32)
    tab_t = table.T
    tail_cols = n % 128
    tail = jnp.pad(tab_t[:, n - tail_cols:], ((0, 0), (0, 128 - tail_cols)))
    tab_r = _make_trans(n)(tab_t, tail)
    out = _make_gather(seq, b)(tab_r, xt)
    return jnp.transpose(out, (2, 0, 1))


# speedup vs baseline: 2.7881x; 2.7881x over previous
import jax
import jax.numpy as jnp
from jax import lax
from jax.experimental import pallas as pl
from jax.experimental.pallas import tpu as pltpu
from jax.experimental.pallas import tpu_sc as plsc

DIM = 32
NC = 2
NS = 16
NW = NC * NS
L = 16
GATHER = 128


def _trans_body(tab_t_hbm, tail_hbm, scr_hbm, in_v, out_v,
                semi0, semi1, semo0, semo1):
    n = tab_t_hbm.shape[1]
    nblk_full = n // 128
    rem = n - nblk_full * 128
    wid = lax.axis_index("s") * NC + lax.axis_index("c")
    npairs = nblk_full // NW // 2
    extra_w = nblk_full - npairs * 2 * NW

    sems_i = (semi0, semi1)
    sems_o = (semo0, semo1)

    def blk(p, buf):
        return wid + (2 * p + buf) * NW

    def start_in(t, buf):
        pltpu.async_copy(tab_t_hbm.at[:, pl.ds(t * 128, 128)], in_v.at[buf],
                         sems_i[buf])

    def drain_in(t, buf):
        pltpu.make_async_copy(tab_t_hbm.at[:, pl.ds(t * 128, 128)],
                              in_v.at[buf], sems_i[buf]).wait()

    iota = jnp.arange(L, dtype=jnp.int32)
    rots = [((iota + r) & (L - 1)) for r in range(L)]

    def transpose(buf, ncolg):
        src = in_v.at[buf]
        dst = out_v.at[buf]

        def wstep(w, carry):
            for ch in range(2):
                cvec = ch * L + iota
                for r in range(L):
                    colv = w * L + rots[r]
                    val = plsc.load_gather(src, [cvec, colv])
                    plsc.store_scatter(
                        dst,
                        [lax.shift_right_logical(colv, 2),
                         (colv & 3) * DIM + cvec],
                        val)
            return carry

        lax.fori_loop(0, ncolg, wstep, 0)

    def start_out(t, buf, nrows):
        pltpu.async_copy(out_v.at[buf, pl.ds(0, nrows)],
                         scr_hbm.at[pl.ds(t * 32, nrows)], sems_o[buf])

    def wait_out(t, buf, nrows):
        pltpu.make_async_copy(out_v.at[buf, pl.ds(0, nrows)],
                              scr_hbm.at[pl.ds(t * 32, nrows)],
                              sems_o[buf]).wait()

    start_in(blk(0, 0), 0)

    def pair(p, carry):
        t0 = blk(p, 0)
        t1 = blk(p, 1)
        start_in(t1, 1)
        drain_in(t0, 0)

        @pl.when(p >= 1)
        def _():
            wait_out(blk(p - 1, 0), 0, 32)

        transpose(0, 8)
        start_out(t0, 0, 32)

        @pl.when(p + 1 < npairs)
        def _():
            start_in(blk(p + 1, 0), 0)

        drain_in(t1, 1)

        @pl.when(p >= 1)
        def _():
            wait_out(blk(p - 1, 1), 1, 32)

        transpose(1, 8)
        start_out(t1, 1, 32)
        return carry

    lax.fori_loop(0, npairs, pair, 0)
    wait_out(blk(npairs - 1, 0), 0, 32)
    wait_out(blk(npairs - 1, 1), 1, 32)

    @pl.when(wid < extra_w)
    def _():
        t = npairs * 2 * NW + wid
        pltpu.sync_copy(tab_t_hbm.at[:, pl.ds(t * 128, 128)], in_v.at[0])
        transpose(0, 8)
        pltpu.sync_copy(out_v.at[0], scr_hbm.at[pl.ds(t * 32, 32)])

    if rem:
        @pl.when(wid == extra_w)
        def _():
            pltpu.sync_copy(tail_hbm, in_v.at[0])
            transpose(0, rem // L)
            pltpu.sync_copy(out_v.at[0, pl.ds(0, rem // 4)],
                            scr_hbm.at[pl.ds(nblk_full * 32, rem // 4)])


def _make_trans(n):
    assert (n * DIM) % 128 == 0
    return pl.kernel(
        _trans_body,
        out_type=jax.ShapeDtypeStruct((n * DIM // 128, 128), jnp.float32),
        mesh=plsc.VectorSubcoreMesh(core_axis_name="c", subcore_axis_name="s"),
        scratch_types=[
            pltpu.VMEM((2, DIM, 128), jnp.float32),
            pltpu.VMEM((2, DIM, 128), jnp.float32),
            pltpu.SemaphoreType.DMA,
            pltpu.SemaphoreType.DMA,
            pltpu.SemaphoreType.DMA,
            pltpu.SemaphoreType.DMA,
        ],
        compiler_params=pltpu.CompilerParams(
            use_tc_tiling_on_sc=True, needs_layout_passes=False),
    )


def _gather_body(tab_hbm, xt_hbm, out_hbm, idxf_v, srow_v, scol_v, rows_v,
                 tile_v, semi, semg0, semg1, semo0, semo1):
    seq = xt_hbm.shape[0]
    bpw = xt_hbm.shape[1] // NW
    upw = seq * (bpw // GATHER)
    wid = lax.axis_index("s") * NC + lax.axis_index("c")
    b0 = wid * bpw

    for s in range(seq):
        pltpu.async_copy(xt_hbm.at[s, pl.ds(b0, bpw)],
                         idxf_v.at[pl.ds(s * bpw, bpw)], semi)
    for s in range(seq):
        pltpu.make_async_copy(xt_hbm.at[s, pl.ds(b0, bpw)],
                              idxf_v.at[pl.ds(s * bpw, bpw)], semi).wait()

    sems_g = (semg0, semg1)
    sems_o = (semo0, semo1)
    nbsub = bpw // GATHER

    def prep(u, buf):
        base = u * GATHER
        for v in range(GATHER // L):
            iv = idxf_v[pl.ds(base + v * L, L)]
            srow_v[buf, pl.ds(v * L, L)] = lax.shift_right_logical(iv, 2)
            scol_v[buf, pl.ds(v * L, L)] = (iv & 3) * DIM

    def fire(buf):
        pltpu.async_copy(tab_hbm.at[srow_v.at[buf]], rows_v.at[buf],
                         sems_g[buf])

    def drain_gather(buf):
        pltpu.make_async_copy(tab_hbm.at[pl.ds(0, GATHER)], rows_v.at[buf],
                              sems_g[buf]).wait()

    iota = jnp.arange(L, dtype=jnp.int32)
    rots = [((iota + r) & (L - 1)) for r in range(L)]

    def extract(buf):
        rows = rows_v.at[buf]
        dst = tile_v.at[buf]

        def vstep(v, carry):
            bvec = v * L + iota
            colbase = scol_v[buf, pl.ds(v * L, L)]
            for ch in range(2):
                for r in range(L):
                    cvec = ch * L + rots[r]
                    val = plsc.load_gather(rows, [bvec, colbase + cvec])
                    plsc.store_scatter(dst, [cvec, bvec], val)
            return carry

        lax.fori_loop(0, GATHER // L, vstep, 0)

    def out_slice(u):
        s = lax.div(u, nbsub)
        bg = b0 + lax.rem(u, nbsub) * GATHER
        return out_hbm.at[s, :, pl.ds(bg, GATHER)]

    def wait_writeout(buf, u):
        pltpu.make_async_copy(tile_v.at[buf], out_slice(u), sems_o[buf]
                              ).wait()

    prep(0, 0)
    fire(0)

    def pair(p, carry):
        u0 = 2 * p
        u1 = 2 * p + 1

        prep(u1, 1)
        drain_gather(0)
        fire(1)

        @pl.when(p >= 1)
        def _():
            wait_writeout(0, u0 - 2)

        extract(0)
        pltpu.async_copy(tile_v.at[0], out_slice(u0), sems_o[0])

        @pl.when(p + 1 < upw // 2)
        def _():
            prep(u0 + 2, 0)
            fire(0)

        drain_gather(1)

        @pl.when(p >= 1)
        def _():
            wait_writeout(1, u1 - 2)

        extract(1)
        pltpu.async_copy(tile_v.at[1], out_slice(u1), sems_o[1])
        return carry

    lax.fori_loop(0, upw // 2, pair, 0)
    wait_writeout(0, upw - 2)
    wait_writeout(1, upw - 1)


def _make_gather(seq, b):
    return pl.kernel(
        _gather_body,
        out_type=jax.ShapeDtypeStruct((seq, DIM, b), jnp.float32),
        mesh=plsc.VectorSubcoreMesh(core_axis_name="c", subcore_axis_name="s"),
        scratch_types=[
            pltpu.VMEM((seq * (b // NW),), jnp.int32),
            pltpu.VMEM((2, GATHER), jnp.int32),
            pltpu.VMEM((2, GATHER), jnp.int32),
            pltpu.VMEM((2, GATHER, 128), jnp.float32),
            pltpu.VMEM((2, DIM, GATHER), jnp.float32),
            pltpu.SemaphoreType.DMA,
            pltpu.SemaphoreType.DMA,
            pltpu.SemaphoreType.DMA,
            pltpu.SemaphoreType.DMA,
            pltpu.SemaphoreType.DMA,
        ],
        compiler_params=pltpu.CompilerParams(
            use_tc_tiling_on_sc=True, needs_layout_passes=False),
    )


def kernel(x, table):
    b, seq = x.shape
    n, d = table.shape
    assert d == DIM and (n * d) % 128 == 0
    xt = x.T.astype(jnp.int32)
    tab_t = table.T
    tail_cols = n % 128
    tail = jnp.pad(tab_t[:, n - tail_cols:], ((0, 0), (0, 128 - tail_cols)))
    tab_r = _make_trans(n)(tab_t, tail)
    out = _make_gather(seq, b)(tab_r, xt)
    return jnp.transpose(out, (2, 0, 1))
